# Initial kernel scaffold; baseline (speedup 1.0000x reference)
import jax
import jax.numpy as jnp
from jax.experimental import pallas as pl


def kernel(x, positions, action, W1, B1, W2, B2, Wp, bp):
    raise NotImplementedError("write your pallas kernel here")


# speedup vs baseline: 1.9051x; 1.9051x over previous
import functools

import jax
import jax.numpy as jnp
from jax import lax
from jax.experimental import pallas as pl

_THR2 = 0.1 * 0.1
_BJ = 512

_INTERPRET = False


def _proj_kernel(x_ref, w_ref, b_ref, y_ref, z_ref):
    xb = x_ref[0].astype(jnp.bfloat16)
    w = w_ref[...].astype(jnp.bfloat16)
    b = b_ref[...].astype(jnp.bfloat16)
    y_ref[0] = lax.dot_general(xb, w, (((1,), (1,)), ((), ())),
                               preferred_element_type=jnp.float32).astype(jnp.bfloat16)
    z_ref[0] = lax.dot_general(xb, b, (((1,), (1,)), ((), ())),
                               preferred_element_type=jnp.float32)


def _masked_agg(posj_ref, post_ref, y_ref, compute_deg):
    jb = pl.program_id(1)
    bj = posj_ref.shape[1]
    h = y_ref.shape[2]

    pj = posj_ref[0]
    sj = jnp.sum(pj * pj, axis=1, keepdims=True)
    aj = 0.5 * sj - 0.5 * _THR2

    def chunk_cond(ic):
        pic = post_ref[0, :, pl.ds(ic * bj, bj)]
        c = lax.dot_general(pj, pic, (((1,), (0,)), ((), ())),
                            preferred_element_type=jnp.float32)
        sic = pic[0:1, :] * pic[0:1, :] + pic[1:2, :] * pic[1:2, :]
        return c >= aj + 0.5 * sic

    def accum(cond, ic, acc, deg):
        mf = jnp.where(cond, 1.0, 0.0)
        mb = mf.astype(jnp.bfloat16)
        ych = y_ref[0, pl.ds(ic * bj, bj), :]
        acc = acc + lax.dot_general(mb, ych, (((1,), (0,)), ((), ())),
                                    preferred_element_type=jnp.float32)
        if compute_deg:
            deg = deg + jnp.sum(mf, axis=1, keepdims=True)
        return acc, deg

    def off_diag(ic, carry):
        acc, deg = carry
        return accum(chunk_cond(ic), ic, acc, deg)

    acc0 = jnp.zeros((bj, h), jnp.float32)
    deg0 = jnp.zeros((bj, 1), jnp.float32)
    acc, deg = lax.fori_loop(0, jb, off_diag, (acc0, deg0))

    tril = (lax.broadcasted_iota(jnp.int32, (bj, bj), 0)
            >= lax.broadcasted_iota(jnp.int32, (bj, bj), 1))
    acc, deg = accum(jnp.logical_and(chunk_cond(jb), tril), jb, acc, deg)
    return acc, (deg if compute_deg else None)


def _activate(acc, inv, z_ref):
    out = jnp.maximum(acc * inv + z_ref[0], 0.0)
    nrm = jnp.sqrt(jnp.sum(out * out, axis=-1, keepdims=True))
    return out / jnp.maximum(nrm, 1e-12)


def _mid_kernel(posj_ref, post_ref, y_ref, z_ref, w2_ref, b2_ref,
                y2_ref, z2_ref, inv_ref):
    acc, deg = _masked_agg(posj_ref, post_ref, y_ref, compute_deg=True)
    inv = 1.0 / jnp.maximum(deg, 1.0)
    inv_ref[0] = inv
    hblk = _activate(acc, inv, z_ref).astype(jnp.bfloat16)
    w2 = w2_ref[...].astype(jnp.bfloat16)
    b2 = b2_ref[...].astype(jnp.bfloat16)
    y2_ref[0] = lax.dot_general(hblk, w2, (((1,), (1,)), ((), ())),
                                preferred_element_type=jnp.float32).astype(jnp.bfloat16)
    z2_ref[0] = lax.dot_general(hblk, b2, (((1,), (1,)), ((), ())),
                                preferred_element_type=jnp.float32)


def _last_kernel(posj_ref, post_ref, y_ref, z_ref, inv_ref, wp_ref, bp_ref,
                 act_ref, lp_ref, ent_ref):
    acc, _ = _masked_agg(posj_ref, post_ref, y_ref, compute_deg=False)
    hblk = _activate(acc, inv_ref[0], z_ref)
    a = wp_ref.shape[0]
    bj = hblk.shape[0]
    logits_t = lax.dot_general(wp_ref[...], hblk, (((1,), (1,)), ((), ())),
                               preferred_element_type=jnp.float32)
    logits_t = logits_t + bp_ref[...]
    m = jnp.max(logits_t, axis=0, keepdims=True)
    ex = jnp.exp(logits_t - m)
    se = jnp.sum(ex, axis=0, keepdims=True)
    logp_t = logits_t - (jnp.log(se) + m)
    act = act_ref[0]
    sel = lax.broadcasted_iota(jnp.int32, (a, bj), 0) == act
    lp_ref[0] = jnp.sum(jnp.where(sel, logp_t, 0.0), axis=0, keepdims=True)
    p = jnp.exp(logp_t)
    ent_ref[0] = -jnp.sum(p * logp_t, axis=0, keepdims=True)


def kernel(x, positions, action, W1, B1, W2, B2, Wp, bp):
    E, N, D = x.shape
    H = W1.shape[0]
    A = Wp.shape[0]
    BJ = _BJ
    JB = N // BJ
    f32 = jnp.float32

    pos_t = jnp.transpose(positions, (0, 2, 1))

    y1, z1 = pl.pallas_call(
        _proj_kernel,
        grid=(E, JB),
        in_specs=[
            pl.BlockSpec((1, BJ, D), lambda e, j: (e, j, 0)),
            pl.BlockSpec((H, D), lambda e, j: (0, 0)),
            pl.BlockSpec((H, D), lambda e, j: (0, 0)),
        ],
        out_specs=[
            pl.BlockSpec((1, BJ, H), lambda e, j: (e, j, 0)),
            pl.BlockSpec((1, BJ, H), lambda e, j: (e, j, 0)),
        ],
        out_shape=[
            jax.ShapeDtypeStruct((E, N, H), jnp.bfloat16),
            jax.ShapeDtypeStruct((E, N, H), f32),
        ],
        interpret=_INTERPRET,
    )(x, W1, B1)

    y2, z2, inv = pl.pallas_call(
        _mid_kernel,
        grid=(E, JB),
        in_specs=[
            pl.BlockSpec((1, BJ, 2), lambda e, j: (e, j, 0)),
            pl.BlockSpec((1, 2, N), lambda e, j: (e, 0, 0)),
            pl.BlockSpec((1, N, H), lambda e, j: (e, 0, 0)),
            pl.BlockSpec((1, BJ, H), lambda e, j: (e, j, 0)),
            pl.BlockSpec((H, H), lambda e, j: (0, 0)),
            pl.BlockSpec((H, H), lambda e, j: (0, 0)),
        ],
        out_specs=[
            pl.BlockSpec((1, BJ, H), lambda e, j: (e, j, 0)),
            pl.BlockSpec((1, BJ, H), lambda e, j: (e, j, 0)),
            pl.BlockSpec((1, BJ, 1), lambda e, j: (e, j, 0)),
        ],
        out_shape=[
            jax.ShapeDtypeStruct((E, N, H), jnp.bfloat16),
            jax.ShapeDtypeStruct((E, N, H), f32),
            jax.ShapeDtypeStruct((E, N, 1), f32),
        ],
        interpret=_INTERPRET,
    )(positions, pos_t, y1, z1, W2, B2)

    act3 = action.reshape(E * JB, 1, BJ)
    lp3, ent3 = pl.pallas_call(
        _last_kernel,
        grid=(E, JB),
        in_specs=[
            pl.BlockSpec((1, BJ, 2), lambda e, j: (e, j, 0)),
            pl.BlockSpec((1, 2, N), lambda e, j: (e, 0, 0)),
            pl.BlockSpec((1, N, H), lambda e, j: (e, 0, 0)),
            pl.BlockSpec((1, BJ, H), lambda e, j: (e, j, 0)),
            pl.BlockSpec((1, BJ, 1), lambda e, j: (e, j, 0)),
            pl.BlockSpec((A, H), lambda e, j: (0, 0)),
            pl.BlockSpec((A, 1), lambda e, j: (0, 0)),
            pl.BlockSpec((1, 1, BJ), lambda e, j, JB=JB: (e * JB + j, 0, 0)),
        ],
        out_specs=[
            pl.BlockSpec((1, 1, BJ), lambda e, j, JB=JB: (e * JB + j, 0, 0)),
            pl.BlockSpec((1, 1, BJ), lambda e, j, JB=JB: (e * JB + j, 0, 0)),
        ],
        out_shape=[
            jax.ShapeDtypeStruct((E * JB, 1, BJ), f32),
            jax.ShapeDtypeStruct((E * JB, 1, BJ), f32),
        ],
        interpret=_INTERPRET,
    )(positions, pos_t, y2, z2, inv, Wp, bp.reshape(A, 1), act3)

    return (action, lp3.reshape(E * N), ent3.reshape(E * N))


# speedup vs baseline: 2.3684x; 1.2432x over previous
import functools

import jax
import jax.numpy as jnp
from jax import lax
from jax.experimental import pallas as pl

_THR2 = 0.1 * 0.1
_BJ = 1024

_INTERPRET = False


def _proj_kernel(x_ref, w_ref, b_ref, y_ref, z_ref):
    xb = x_ref[0].astype(jnp.bfloat16)
    w = w_ref[...].astype(jnp.bfloat16)
    b = b_ref[...].astype(jnp.bfloat16)
    y_ref[0] = lax.dot_general(xb, w, (((1,), (1,)), ((), ())),
                               preferred_element_type=jnp.float32).astype(jnp.bfloat16)
    z_ref[0] = lax.dot_general(xb, b, (((1,), (1,)), ((), ())),
                               preferred_element_type=jnp.float32)


def _masked_agg(posj_ref, post_ref, y_ref, compute_deg):
    jb = pl.program_id(1)
    bj = posj_ref.shape[1]
    h = y_ref.shape[2]

    pj = posj_ref[0]
    sj = jnp.sum(pj * pj, axis=1, keepdims=True)
    aj = 0.5 * sj - 0.5 * _THR2

    def chunk_cond(ic):
        pic = post_ref[0, :, pl.ds(ic * bj, bj)]
        c = lax.dot_general(pj, pic, (((1,), (0,)), ((), ())),
                            preferred_element_type=jnp.float32)
        sic = pic[0:1, :] * pic[0:1, :] + pic[1:2, :] * pic[1:2, :]
        return c >= aj + 0.5 * sic

    def accum(cond, ic, acc, deg):
        mf = jnp.where(cond, 1.0, 0.0)
        mb = mf.astype(jnp.bfloat16)
        if compute_deg:
            deg = deg + jnp.sum(mf, axis=1, keepdims=True)
        ych = y_ref[0, pl.ds(ic * bj, bj), :]
        acc = acc + lax.dot_general(mb, ych, (((1,), (0,)), ((), ())),
                                    preferred_element_type=jnp.float32)
        return acc, deg

    def off_diag(ic, carry):
        acc, deg = carry
        return accum(chunk_cond(ic), ic, acc, deg)

    acc0 = jnp.zeros((bj, h), jnp.float32)
    deg0 = jnp.zeros((bj, 1), jnp.float32)
    acc, deg = lax.fori_loop(0, jb, off_diag, (acc0, deg0))

    tril = (lax.broadcasted_iota(jnp.int32, (bj, bj), 0)
            >= lax.broadcasted_iota(jnp.int32, (bj, bj), 1))
    acc, deg = accum(jnp.logical_and(chunk_cond(jb), tril), jb, acc, deg)
    return acc, (deg if compute_deg else None)


def _activate(acc, inv, z_ref):
    out = jnp.maximum(acc * inv + z_ref[0], 0.0)
    n2 = jnp.sum(out * out, axis=-1, keepdims=True)
    return out * lax.rsqrt(jnp.maximum(n2, 1e-24))


def _mid_kernel(posj_ref, post_ref, y_ref, z_ref, w2_ref, b2_ref,
                y2_ref, z2_ref, inv_ref):
    acc, deg = _masked_agg(posj_ref, post_ref, y_ref, compute_deg=True)
    inv = 1.0 / jnp.maximum(deg, 1.0)
    inv_ref[0] = inv
    hblk = _activate(acc, inv, z_ref).astype(jnp.bfloat16)
    w2 = w2_ref[...].astype(jnp.bfloat16)
    b2 = b2_ref[...].astype(jnp.bfloat16)
    y2_ref[0] = lax.dot_general(hblk, w2, (((1,), (1,)), ((), ())),
                                preferred_element_type=jnp.float32).astype(jnp.bfloat16)
    z2_ref[0] = lax.dot_general(hblk, b2, (((1,), (1,)), ((), ())),
                                preferred_element_type=jnp.float32)


def _last_kernel(posj_ref, post_ref, y_ref, z_ref, inv_ref, wp_ref, bp_ref,
                 act_ref, lp_ref, ent_ref):
    acc, _ = _masked_agg(posj_ref, post_ref, y_ref, compute_deg=False)
    hblk = _activate(acc, inv_ref[0], z_ref)
    a = wp_ref.shape[0]
    bj = hblk.shape[0]
    logits_t = lax.dot_general(wp_ref[...], hblk, (((1,), (1,)), ((), ())),
                               preferred_element_type=jnp.float32)
    logits_t = logits_t + bp_ref[...]
    m = jnp.max(logits_t, axis=0, keepdims=True)
    ex = jnp.exp(logits_t - m)
    se = jnp.sum(ex, axis=0, keepdims=True)
    logp_t = logits_t - (jnp.log(se) + m)
    act = act_ref[0]
    sel = lax.broadcasted_iota(jnp.int32, (a, bj), 0) == act
    lp_ref[0] = jnp.sum(jnp.where(sel, logp_t, 0.0), axis=0, keepdims=True)
    p = jnp.exp(logp_t)
    ent_ref[0] = -jnp.sum(p * logp_t, axis=0, keepdims=True)


def kernel(x, positions, action, W1, B1, W2, B2, Wp, bp):
    E, N, D = x.shape
    H = W1.shape[0]
    A = Wp.shape[0]
    BJ = _BJ
    JB = N // BJ
    f32 = jnp.float32

    pos_t = jnp.transpose(positions, (0, 2, 1))

    y1, z1 = pl.pallas_call(
        _proj_kernel,
        grid=(E, JB),
        in_specs=[
            pl.BlockSpec((1, BJ, D), lambda e, j: (e, j, 0)),
            pl.BlockSpec((H, D), lambda e, j: (0, 0)),
            pl.BlockSpec((H, D), lambda e, j: (0, 0)),
        ],
        out_specs=[
            pl.BlockSpec((1, BJ, H), lambda e, j: (e, j, 0)),
            pl.BlockSpec((1, BJ, H), lambda e, j: (e, j, 0)),
        ],
        out_shape=[
            jax.ShapeDtypeStruct((E, N, H), jnp.bfloat16),
            jax.ShapeDtypeStruct((E, N, H), f32),
        ],
        interpret=_INTERPRET,
    )(x, W1, B1)

    y2, z2, inv = pl.pallas_call(
        _mid_kernel,
        grid=(E, JB),
        in_specs=[
            pl.BlockSpec((1, BJ, 2), lambda e, j: (e, j, 0)),
            pl.BlockSpec((1, 2, N), lambda e, j: (e, 0, 0)),
            pl.BlockSpec((1, N, H), lambda e, j: (e, 0, 0)),
            pl.BlockSpec((1, BJ, H), lambda e, j: (e, j, 0)),
            pl.BlockSpec((H, H), lambda e, j: (0, 0)),
            pl.BlockSpec((H, H), lambda e, j: (0, 0)),
        ],
        out_specs=[
            pl.BlockSpec((1, BJ, H), lambda e, j: (e, j, 0)),
            pl.BlockSpec((1, BJ, H), lambda e, j: (e, j, 0)),
            pl.BlockSpec((1, BJ, 1), lambda e, j: (e, j, 0)),
        ],
        out_shape=[
            jax.ShapeDtypeStruct((E, N, H), jnp.bfloat16),
            jax.ShapeDtypeStruct((E, N, H), f32),
            jax.ShapeDtypeStruct((E, N, 1), f32),
        ],
        interpret=_INTERPRET,
    )(positions, pos_t, y1, z1, W2, B2)

    act3 = action.reshape(E * JB, 1, BJ)
    lp3, ent3 = pl.pallas_call(
        _last_kernel,
        grid=(E, JB),
        in_specs=[
            pl.BlockSpec((1, BJ, 2), lambda e, j: (e, j, 0)),
            pl.BlockSpec((1, 2, N), lambda e, j: (e, 0, 0)),
            pl.BlockSpec((1, N, H), lambda e, j: (e, 0, 0)),
            pl.BlockSpec((1, BJ, H), lambda e, j: (e, j, 0)),
            pl.BlockSpec((1, BJ, 1), lambda e, j: (e, j, 0)),
            pl.BlockSpec((A, H), lambda e, j: (0, 0)),
            pl.BlockSpec((A, 1), lambda e, j: (0, 0)),
            pl.BlockSpec((1, 1, BJ), lambda e, j, JB=JB: (e * JB + j, 0, 0)),
        ],
        out_specs=[
            pl.BlockSpec((1, 1, BJ), lambda e, j, JB=JB: (e * JB + j, 0, 0)),
            pl.BlockSpec((1, 1, BJ), lambda e, j, JB=JB: (e * JB + j, 0, 0)),
        ],
        out_shape=[
            jax.ShapeDtypeStruct((E * JB, 1, BJ), f32),
            jax.ShapeDtypeStruct((E * JB, 1, BJ), f32),
        ],
        interpret=_INTERPRET,
    )(positions, pos_t, y2, z2, inv, Wp, bp.reshape(A, 1), act3)

    return (action, lp3.reshape(E * N), ent3.reshape(E * N))


# speedup vs baseline: 3.0809x; 1.3008x over previous
import jax
import jax.numpy as jnp
from jax import lax
from jax.experimental import pallas as pl
from jax.experimental.pallas import tpu as pltpu

_THR2 = 0.1 * 0.1
_BJ = 1024

_INTERPRET = False


def _bdot(a, b, dims):
    return lax.dot_general(a, b, (dims, ((), ())),
                           preferred_element_type=jnp.float32)


def _activate(acc, inv, z):
    out = jnp.maximum(acc * inv + z, 0.0)
    n2 = jnp.sum(out * out, axis=-1, keepdims=True)
    return out * lax.rsqrt(jnp.maximum(n2, 1e-24))


def _fused_kernel(posj_ref, post_ref, x_ref, w1_ref, b1_ref, w2_ref, b2_ref,
                  wp_ref, bp_ref, act_ref, lp_ref, ent_ref,
                  y1_scr, y2_scr, msk_scr):
    jb = pl.program_id(1)
    bj = posj_ref.shape[1]
    h = w1_ref.shape[0]
    a = wp_ref.shape[0]
    f32 = jnp.float32
    bf16 = jnp.bfloat16

    xb = x_ref[0].astype(bf16)
    y1c = _bdot(xb, w1_ref[...].astype(bf16), ((1,), (1,))).astype(bf16)
    y1_scr[pl.ds(jb * bj, bj), :] = y1c
    z1c = _bdot(xb, b1_ref[...].astype(bf16), ((1,), (1,)))

    pj = posj_ref[0]
    sj = jnp.sum(pj * pj, axis=1, keepdims=True)
    aj = 0.5 * sj - 0.5 * _THR2

    def chunk_cond(ic):
        pic = post_ref[0, :, pl.ds(ic * bj, bj)]
        c = _bdot(pj, pic, ((1,), (0,)))
        sic = pic[0:1, :] * pic[0:1, :] + pic[1:2, :] * pic[1:2, :]
        return c >= aj + 0.5 * sic

    def build(cond, ic, acc, deg):
        mf = jnp.where(cond, 1.0, 0.0)
        mb = mf.astype(bf16)
        msk_scr[:, pl.ds(ic * bj, bj)] = mb
        deg = deg + jnp.sum(mf, axis=1, keepdims=True)
        acc = acc + _bdot(mb, y1_scr[pl.ds(ic * bj, bj), :], ((1,), (0,)))
        return acc, deg

    def off_diag(ic, carry):
        acc, deg = carry
        return build(chunk_cond(ic), ic, acc, deg)

    acc0 = jnp.zeros((bj, h), f32)
    deg0 = jnp.zeros((bj, 1), f32)
    acc1, deg = lax.fori_loop(0, jb, off_diag, (acc0, deg0))
    tril = (lax.broadcasted_iota(jnp.int32, (bj, bj), 0)
            >= lax.broadcasted_iota(jnp.int32, (bj, bj), 1))
    acc1, deg = build(jnp.logical_and(chunk_cond(jb), tril), jb, acc1, deg)

    inv = 1.0 / jnp.maximum(deg, 1.0)

    h1 = _activate(acc1, inv, z1c).astype(bf16)
    y2c = _bdot(h1, w2_ref[...].astype(bf16), ((1,), (1,))).astype(bf16)
    y2_scr[pl.ds(jb * bj, bj), :] = y2c
    z2c = _bdot(h1, b2_ref[...].astype(bf16), ((1,), (1,)))

    def agg2(ic, acc):
        mb = msk_scr[:, pl.ds(ic * bj, bj)]
        return acc + _bdot(mb, y2_scr[pl.ds(ic * bj, bj), :], ((1,), (0,)))

    acc2 = lax.fori_loop(0, jb + 1, agg2, jnp.zeros((bj, h), f32))
    h2 = _activate(acc2, inv, z2c)

    logits_t = _bdot(wp_ref[...], h2, ((1,), (1,))) + bp_ref[...]
    m = jnp.max(logits_t, axis=0, keepdims=True)
    ex = jnp.exp(logits_t - m)
    se = jnp.sum(ex, axis=0, keepdims=True)
    logp_t = logits_t - (jnp.log(se) + m)
    act = act_ref[0]
    sel = lax.broadcasted_iota(jnp.int32, (a, bj), 0) == act
    lp_ref[0] = jnp.sum(jnp.where(sel, logp_t, 0.0), axis=0, keepdims=True)
    p = jnp.exp(logp_t)
    ent_ref[0] = -jnp.sum(p * logp_t, axis=0, keepdims=True)


def kernel(x, positions, action, W1, B1, W2, B2, Wp, bp):
    E, N, D = x.shape
    H = W1.shape[0]
    A = Wp.shape[0]
    BJ = _BJ
    JB = N // BJ
    f32 = jnp.float32

    pos_t = jnp.transpose(positions, (0, 2, 1))
    act3 = action.reshape(E * JB, 1, BJ)

    lp3, ent3 = pl.pallas_call(
        _fused_kernel,
        grid=(E, JB),
        in_specs=[
            pl.BlockSpec((1, BJ, 2), lambda e, j: (e, j, 0)),
            pl.BlockSpec((1, 2, N), lambda e, j: (e, 0, 0)),
            pl.BlockSpec((1, BJ, D), lambda e, j: (e, j, 0)),
            pl.BlockSpec((H, D), lambda e, j: (0, 0)),
            pl.BlockSpec((H, D), lambda e, j: (0, 0)),
            pl.BlockSpec((H, H), lambda e, j: (0, 0)),
            pl.BlockSpec((H, H), lambda e, j: (0, 0)),
            pl.BlockSpec((A, H), lambda e, j: (0, 0)),
            pl.BlockSpec((A, 1), lambda e, j: (0, 0)),
            pl.BlockSpec((1, 1, BJ), lambda e, j, JB=JB: (e * JB + j, 0, 0)),
        ],
        out_specs=[
            pl.BlockSpec((1, 1, BJ), lambda e, j, JB=JB: (e * JB + j, 0, 0)),
            pl.BlockSpec((1, 1, BJ), lambda e, j, JB=JB: (e * JB + j, 0, 0)),
        ],
        out_shape=[
            jax.ShapeDtypeStruct((E * JB, 1, BJ), f32),
            jax.ShapeDtypeStruct((E * JB, 1, BJ), f32),
        ],
        scratch_shapes=[
            pltpu.VMEM((N, H), jnp.bfloat16),
            pltpu.VMEM((N, H), jnp.bfloat16),
            pltpu.VMEM((BJ, N), jnp.bfloat16),
        ],
        interpret=_INTERPRET,
    )(positions, pos_t, x, W1, B1, W2, B2, Wp, bp.reshape(A, 1), act3)

    return (action, lp3.reshape(E * N), ent3.reshape(E * N))


# speedup vs baseline: 3.0996x; 1.0060x over previous
import jax
import jax.numpy as jnp
from jax import lax
from jax.experimental import pallas as pl
from jax.experimental.pallas import tpu as pltpu

_THR2 = 0.1 * 0.1
_BJ = 1024

_INTERPRET = False


def _bdot(a, b, dims):
    return lax.dot_general(a, b, (dims, ((), ())),
                           preferred_element_type=jnp.float32)


def _activate(acc, inv, z):
    out = jnp.maximum(acc * inv + z, 0.0)
    n2 = jnp.sum(out * out, axis=-1, keepdims=True)
    return out * lax.rsqrt(jnp.maximum(n2, 1e-24))


def _fused_kernel(posj_ref, post_ref, x_ref, w1_ref, b1_ref, w2_ref, b2_ref,
                  wp_ref, bp_ref, act_ref, lp_ref, ent_ref,
                  y1_scr, y2_scr, msk_scr):
    jb = pl.program_id(1)
    bj = posj_ref.shape[1]
    h = w1_ref.shape[0]
    a = wp_ref.shape[0]
    f32 = jnp.float32
    bf16 = jnp.bfloat16

    xb = x_ref[0].astype(bf16)
    y1c = _bdot(xb, w1_ref[...].astype(bf16), ((1,), (1,))).astype(bf16)
    y1_scr[pl.ds(jb * bj, bj), :] = y1c
    z1c = _bdot(xb, b1_ref[...].astype(bf16), ((1,), (1,)))

    pj = posj_ref[0]
    sj = jnp.sum(pj * pj, axis=1, keepdims=True)
    aj = 0.5 * sj - 0.5 * _THR2
    pj4 = jnp.concatenate([pj, jnp.ones((bj, 1), f32), -aj], axis=1)

    def chunk_cond(ic):
        pic = post_ref[0, :, pl.ds(ic * bj, bj)]
        sic = pic[0:1, :] * pic[0:1, :] + pic[1:2, :] * pic[1:2, :]
        pic4 = jnp.concatenate([pic, -0.5 * sic, jnp.ones((1, bj), f32)],
                               axis=0)
        return _bdot(pj4, pic4, ((1,), (0,))) >= 0.0

    def build(cond, ic, acc, deg):
        mf = jnp.where(cond, 1.0, 0.0)
        mb = mf.astype(bf16)
        msk_scr[:, pl.ds(ic * bj, bj)] = mb
        deg = deg + jnp.sum(mf, axis=1, keepdims=True)
        acc = acc + _bdot(mb, y1_scr[pl.ds(ic * bj, bj), :], ((1,), (0,)))
        return acc, deg

    def off_diag(ic, carry):
        acc, deg = carry
        return build(chunk_cond(ic), ic, acc, deg)

    acc0 = jnp.zeros((bj, h), f32)
    deg0 = jnp.zeros((bj, 1), f32)
    acc1, deg = lax.fori_loop(0, jb, off_diag, (acc0, deg0))
    tril = (lax.broadcasted_iota(jnp.int32, (bj, bj), 0)
            >= lax.broadcasted_iota(jnp.int32, (bj, bj), 1))
    acc1, deg = build(jnp.logical_and(chunk_cond(jb), tril), jb, acc1, deg)

    inv = 1.0 / jnp.maximum(deg, 1.0)

    h1 = _activate(acc1, inv, z1c).astype(bf16)
    y2c = _bdot(h1, w2_ref[...].astype(bf16), ((1,), (1,))).astype(bf16)
    y2_scr[pl.ds(jb * bj, bj), :] = y2c
    z2c = _bdot(h1, b2_ref[...].astype(bf16), ((1,), (1,)))

    def agg2(ic, acc):
        mb = msk_scr[:, pl.ds(ic * bj, bj)]
        return acc + _bdot(mb, y2_scr[pl.ds(ic * bj, bj), :], ((1,), (0,)))

    acc2 = lax.fori_loop(0, jb + 1, agg2, jnp.zeros((bj, h), f32))
    h2 = _activate(acc2, inv, z2c)

    logits_t = _bdot(wp_ref[...], h2, ((1,), (1,))) + bp_ref[...]
    m = jnp.max(logits_t, axis=0, keepdims=True)
    ex = jnp.exp(logits_t - m)
    se = jnp.sum(ex, axis=0, keepdims=True)
    logp_t = logits_t - (jnp.log(se) + m)
    act = act_ref[0]
    sel = lax.broadcasted_iota(jnp.int32, (a, bj), 0) == act
    lp_ref[0] = jnp.sum(jnp.where(sel, logp_t, 0.0), axis=0, keepdims=True)
    p = jnp.exp(logp_t)
    ent_ref[0] = -jnp.sum(p * logp_t, axis=0, keepdims=True)


def kernel(x, positions, action, W1, B1, W2, B2, Wp, bp):
    E, N, D = x.shape
    H = W1.shape[0]
    A = Wp.shape[0]
    BJ = _BJ
    JB = N // BJ
    f32 = jnp.float32

    pos_t = jnp.transpose(positions, (0, 2, 1))
    act3 = action.reshape(E * JB, 1, BJ)

    lp3, ent3 = pl.pallas_call(
        _fused_kernel,
        grid=(E, JB),
        in_specs=[
            pl.BlockSpec((1, BJ, 2), lambda e, j: (e, j, 0)),
            pl.BlockSpec((1, 2, N), lambda e, j: (e, 0, 0)),
            pl.BlockSpec((1, BJ, D), lambda e, j: (e, j, 0)),
            pl.BlockSpec((H, D), lambda e, j: (0, 0)),
            pl.BlockSpec((H, D), lambda e, j: (0, 0)),
            pl.BlockSpec((H, H), lambda e, j: (0, 0)),
            pl.BlockSpec((H, H), lambda e, j: (0, 0)),
            pl.BlockSpec((A, H), lambda e, j: (0, 0)),
            pl.BlockSpec((A, 1), lambda e, j: (0, 0)),
            pl.BlockSpec((1, 1, BJ), lambda e, j, JB=JB: (e * JB + j, 0, 0)),
        ],
        out_specs=[
            pl.BlockSpec((1, 1, BJ), lambda e, j, JB=JB: (e * JB + j, 0, 0)),
            pl.BlockSpec((1, 1, BJ), lambda e, j, JB=JB: (e * JB + j, 0, 0)),
        ],
        out_shape=[
            jax.ShapeDtypeStruct((E * JB, 1, BJ), f32),
            jax.ShapeDtypeStruct((E * JB, 1, BJ), f32),
        ],
        scratch_shapes=[
            pltpu.VMEM((N, H), jnp.bfloat16),
            pltpu.VMEM((N, H), jnp.bfloat16),
            pltpu.VMEM((BJ, N), jnp.bfloat16),
        ],
        interpret=_INTERPRET,
    )(positions, pos_t, x, W1, B1, W2, B2, Wp, bp.reshape(A, 1), act3)

    return (action, lp3.reshape(E * N), ent3.reshape(E * N))
